# baseline (speedup 1.0000x reference)
import functools

import jax
import jax.numpy as jnp
from jax import lax
from jax.experimental import pallas as pl
from jax.experimental.pallas import tpu as pltpu
from jax.experimental.pallas import tpu_sc as plsc

NC = 2
NS = 16
L = 16
NW = NC * NS

B = 16384
D = 64
USER_N = 100000
ITEM_N = 1000000
U_BANDS = (USER_N + 127) // 128
I_BANDS = (ITEM_N + 127) // 128
U_PB = -(-U_BANDS // NW)
I_PB = -(-I_BANDS // NW)
U_PAD = U_BANDS * 128
CHUNK = 4096
PMAX = 2048
WIN = 2

_CP = pltpu.CompilerParams(needs_layout_passes=False,
                           use_tc_tiling_on_sc=True)


def _detile_user_body(ut_hbm, ustage_hbm, bb, st, sem):
    wid = lax.axis_index("s") * NC + lax.axis_index("c")
    lo = wid * U_PB
    hi = jnp.minimum(lo + U_PB, U_BANDS)
    lane = lax.iota(jnp.int32, L)

    def band_body(c, _):
        @pl.when(c < hi)
        def _():
            off = pl.multiple_of(c * 128, 128)
            pltpu.sync_copy(ut_hbm.at[:, pl.ds(off, 128)], bb)
            for d in range(D):
                col = jnp.full((L,), d, jnp.int32)
                for p in range(8):
                    v = bb[d, pl.ds(p * L, L)]
                    plsc.store_scatter(st, [p * L + lane, col], v)
            pltpu.sync_copy(st, ustage_hbm.at[pl.ds(off, 128), :])
        return 0

    lax.fori_loop(lo, lo + U_PB, band_body, 0)


def _item_body(user_hbm, item_i_hbm, item_j_hbm, it_hbm, ustage_hbm,
               pred_hbm,
               uid_all, ichunk, hb, hidx, pb, pcol, bb, urow, res, sem):
    wid = lax.axis_index("s") * NC + lax.axis_index("c")
    lo = wid * I_PB
    hi = jnp.minimum(lo + I_PB, I_BANDS)
    lane = lax.iota(jnp.int32, L)

    pltpu.sync_copy(user_hbm, uid_all)

    def scan_stream(src_hbm, boff, ptr0):
        def chunk_body(k, ptr):
            pltpu.sync_copy(src_hbm.at[pl.ds(k * CHUNK, CHUNK)], ichunk)

            def vec_body(q, ptr):
                iv = ichunk[pl.ds(q * L, L)]
                band = iv >> 7
                m = (band >= lo) & (band < hi)
                bv = boff + k * CHUNK + q * L + lane
                plsc.store_compressed(hb.at[pl.ds(ptr, L)], bv, mask=m)
                plsc.store_compressed(hidx.at[pl.ds(ptr, L)], iv, mask=m)
                return ptr + plsc.all_reduce_population_count(m)[0]

            return lax.fori_loop(0, CHUNK // L, vec_body, ptr)

        return lax.fori_loop(0, B // CHUNK, chunk_body, ptr0)

    ptr = scan_stream(item_i_hbm, 0, jnp.int32(0))
    ptr = scan_stream(item_j_hbm, B, ptr)
    nvec = (ptr + L - 1) // L

    def window_body(w, _):
        c0 = lo + w * WIN
        cend = jnp.minimum(c0 + WIN, hi)

        @pl.when(c0 < hi)
        def _():
            def pack_body(q, p2):
                ok = (q * L + lane) < ptr
                iv = hidx[pl.ds(q * L, L)]
                bv = hb[pl.ds(q * L, L)]
                band = iv >> 7
                m = ok & (band >= c0) & (band < cend)
                scol = (band - c0) * 128 + (iv & 127)
                p2c = jnp.minimum(p2, PMAX)
                plsc.store_compressed(pb.at[pl.ds(p2c, L)], bv, mask=m)
                plsc.store_compressed(pcol.at[pl.ds(p2c, L)], scol, mask=m)
                return p2 + plsc.all_reduce_population_count(m)[0]

            p2 = jnp.minimum(lax.fori_loop(0, nvec, pack_body, jnp.int32(0)),
                             PMAX)

            @pl.when(p2 > 0)
            def _():
                plsc.store_compressed(pb.at[pl.ds(p2, L)],
                                      jnp.full((L,), -1, jnp.int32),
                                      mask=lane < L)
                off = pl.multiple_of(c0 * 128, 128)
                pltpu.sync_copy(it_hbm.at[:, pl.ds(off, 128)],
                                bb.at[:, pl.ds(0, 128)])

                @pl.when(c0 + 1 < hi)
                def _():
                    off2 = pl.multiple_of((c0 + 1) * 128, 128)
                    pltpu.sync_copy(it_hbm.at[:, pl.ds(off2, 128)],
                                    bb.at[:, pl.ds(128, 128)])

                def group_body(g, _):
                    ev = pb[pl.ds(g * L, L)]
                    scol = pcol[pl.ds(g * L, L)]
                    valid = ev >= 0
                    rows = ev & (B - 1)
                    uid = plsc.load_gather(uid_all, [rows], mask=valid)
                    uid = jnp.where(valid, uid, 0)
                    pltpu.async_copy(ustage_hbm.at[uid], urow, sem).wait()
                    acc = jnp.zeros((L,), jnp.float32)
                    for d in range(D):
                        dcol = jnp.full((L,), d, jnp.int32)
                        iv_d = plsc.load_gather(bb, [dcol, scol], mask=valid)
                        u_d = plsc.load_gather(urow, [lane, dcol])
                        acc = acc + iv_d * u_d
                    res[pl.ds(0, L)] = acc
                    pltpu.async_copy(
                        res,
                        pred_hbm.at[plsc.Indices(ev, ignored_value=-1)],
                        sem).wait()
                    return 0

                lax.fori_loop(0, (p2 + L - 1) // L, group_body, 0)

        return 0

    lax.fori_loop(0, -(-I_PB // WIN), window_body, 0)


@jax.jit
def _bpr(user, item_i, item_j, embed_user_weight, embed_item_weight):
    mesh = plsc.VectorSubcoreMesh(core_axis_name="c", subcore_axis_name="s",
                                  num_cores=NC, num_subcores=NS)
    k1 = functools.partial(
        pl.kernel,
        out_type=jax.ShapeDtypeStruct((U_PAD, 128), jnp.float32),
        mesh=mesh,
        compiler_params=_CP,
        scratch_types=[
            pltpu.VMEM((D, 128), jnp.float32),
            pltpu.VMEM((128, 128), jnp.float32),
            pltpu.SemaphoreType.DMA,
        ],
    )(_detile_user_body)
    ustage = k1(embed_user_weight.T)

    k2 = functools.partial(
        pl.kernel,
        out_type=jax.ShapeDtypeStruct((2 * B,), jnp.float32),
        mesh=mesh,
        compiler_params=_CP,
        scratch_types=[
            pltpu.VMEM((B,), jnp.int32),
            pltpu.VMEM((CHUNK,), jnp.int32),
            pltpu.VMEM((2 * B + L,), jnp.int32),
            pltpu.VMEM((2 * B + L,), jnp.int32),
            pltpu.VMEM((PMAX + 2 * L,), jnp.int32),
            pltpu.VMEM((PMAX + 2 * L,), jnp.int32),
            pltpu.VMEM((D, 256), jnp.float32),
            pltpu.VMEM((L, 128), jnp.float32),
            pltpu.VMEM((L,), jnp.float32),
            pltpu.SemaphoreType.DMA,
        ],
    )(_item_body)
    pred = k2(user, item_i, item_j, embed_item_weight.T, ustage)
    return pred[:B], pred[B:]


def kernel(user, item_i, item_j, embed_user_weight, embed_item_weight):
    return _bpr(user, item_i, item_j, embed_user_weight, embed_item_weight)
